# baseline (speedup 1.0000x reference)
import functools

import jax
import jax.numpy as jnp
from jax import lax
from jax.experimental import pallas as pl
from jax.experimental.pallas import tpu as pltpu
from jax.experimental.pallas import tpu_sc as plsc


def _emb_call(ids_flat, word_emb, pos_emb, seq_len):
    n_rows, = ids_flat.shape
    _, d = word_emb.shape

    info = plsc.get_sparse_core_info()
    nc, ns, lanes = info.num_cores, info.num_subcores, info.num_lanes
    nw = nc * ns
    b_per_w = n_rows // nw
    chunk = 32
    n_chunks = b_per_w // chunk

    mesh = plsc.VectorSubcoreMesh(core_axis_name="c", subcore_axis_name="s")

    @functools.partial(
        pl.kernel,
        mesh=mesh,
        out_type=jax.ShapeDtypeStruct((n_rows, d), jnp.float32),
        scratch_types=[
            pltpu.VMEM((b_per_w,), jnp.int32),
            pltpu.VMEM((chunk, d), jnp.float32),
            pltpu.VMEM((chunk, d), jnp.float32),
            pltpu.SemaphoreType.DMA,
        ],
    )
    def emb_kernel(ids_hbm, wemb_hbm, pemb_hbm, out_hbm, idx_v, rows_v, pos_v, sem):
        wid = lax.axis_index("s") * nc + lax.axis_index("c")
        base = wid * b_per_w
        pos_base = lax.rem(base, seq_len)
        pltpu.sync_copy(ids_hbm.at[pl.ds(base, b_per_w)], idx_v)
        for c in range(n_chunks):
            pltpu.async_copy(
                wemb_hbm.at[idx_v.at[pl.ds(c * chunk, chunk)]], rows_v, sem
            ).wait()
            pltpu.sync_copy(pemb_hbm.at[pl.ds(pos_base + c * chunk, chunk)], pos_v)

            def body(r, carry):
                for j in range(d // lanes):
                    sl = pl.ds(j * lanes, lanes)
                    plsc.addupdate(rows_v.at[r, sl], pos_v[r, sl])
                return carry

            lax.fori_loop(0, chunk, body, 0)
            pltpu.sync_copy(rows_v, out_hbm.at[pl.ds(base + c * chunk, chunk)])

    return emb_kernel(ids_flat, word_emb, pos_emb)


def kernel(input_ids, word_emb, pos_emb):
    b, s = input_ids.shape
    _, d = word_emb.shape
    ids_flat = input_ids.reshape(b * s).astype(jnp.int32)
    out = _emb_call(ids_flat, word_emb, pos_emb, s)
    return out.reshape(b, s, d)


# speedup vs baseline: 1.3927x; 1.3927x over previous
import functools

import jax
import jax.numpy as jnp
from jax import lax
from jax.experimental import pallas as pl
from jax.experimental.pallas import tpu as pltpu
from jax.experimental.pallas import tpu_sc as plsc

_CHUNK = 16
_NBUF = 3


def _emb_call(ids_flat, word_emb, pos_emb, seq_len):
    n_rows, = ids_flat.shape
    _, d = word_emb.shape

    info = plsc.get_sparse_core_info()
    nc, ns, lanes = info.num_cores, info.num_subcores, info.num_lanes
    nw = nc * ns
    b_per_w = n_rows // nw
    chunk = _CHUNK
    n_chunks = b_per_w // chunk
    nbuf = _NBUF

    mesh = plsc.VectorSubcoreMesh(core_axis_name="c", subcore_axis_name="s")

    scratch = [pltpu.VMEM((b_per_w,), jnp.int32)]
    scratch += [pltpu.VMEM((chunk, d), jnp.float32) for _ in range(2 * nbuf)]
    scratch += [pltpu.SemaphoreType.DMA for _ in range(3 * nbuf)]

    @functools.partial(
        pl.kernel,
        mesh=mesh,
        out_type=jax.ShapeDtypeStruct((n_rows, d), jnp.float32),
        scratch_types=scratch,
    )
    def emb_kernel(ids_hbm, wemb_hbm, pemb_hbm, out_hbm, idx_v, *bufs):
        rows = bufs[:nbuf]
        pos = bufs[nbuf:2 * nbuf]
        gsem = bufs[2 * nbuf:3 * nbuf]
        psem = bufs[3 * nbuf:4 * nbuf]
        osem = bufs[4 * nbuf:5 * nbuf]

        wid = lax.axis_index("s") * nc + lax.axis_index("c")
        base = wid * b_per_w
        pos_base = lax.rem(base, seq_len)
        pltpu.sync_copy(ids_hbm.at[pl.ds(base, b_per_w)], idx_v)

        g_handles = [None] * n_chunks
        p_handles = [None] * n_chunks
        o_handles = [None] * n_chunks

        def start(c):
            b = c % nbuf
            g_handles[c] = pltpu.async_copy(
                wemb_hbm.at[idx_v.at[pl.ds(c * chunk, chunk)]], rows[b], gsem[b]
            )
            p_handles[c] = pltpu.async_copy(
                pemb_hbm.at[pl.ds(pos_base + c * chunk, chunk)], pos[b], psem[b]
            )

        def process(c):
            b = c % nbuf
            g_handles[c].wait()
            p_handles[c].wait()

            def body(r, carry):
                for j in range(d // lanes):
                    sl = pl.ds(j * lanes, lanes)
                    plsc.addupdate(rows[b].at[r, sl], pos[b][r, sl])
                return carry

            lax.fori_loop(0, chunk, body, 0)
            o_handles[c] = pltpu.async_copy(
                rows[b], out_hbm.at[pl.ds(base + c * chunk, chunk)], osem[b]
            )

        for c in range(n_chunks):
            if c >= nbuf:
                o_handles[c - nbuf].wait()
            start(c)
            if c >= 1:
                process(c - 1)
        process(n_chunks - 1)
        for c in range(n_chunks - nbuf, n_chunks):
            o_handles[c].wait()

    return emb_kernel(ids_flat, word_emb, pos_emb)


def kernel(input_ids, word_emb, pos_emb):
    b, s = input_ids.shape
    _, d = word_emb.shape
    ids_flat = input_ids.reshape(b * s).astype(jnp.int32)
    out = _emb_call(ids_flat, word_emb, pos_emb, s)
    return out.reshape(b, s, d)
